# baseline (speedup 1.0000x reference)
import functools

import jax
import jax.numpy as jnp
from jax import lax
from jax.experimental import pallas as pl
from jax.experimental.pallas import tpu as pltpu
from jax.experimental.pallas import tpu_sc as plsc

D = 128
DGW = 16
NC = 2
NS = 16
CH = 128
BN = 1000


def _sc_segment_sums(x, edges, n_acc, kpt):
    @functools.partial(
        pl.kernel,
        out_type=(jax.ShapeDtypeStruct((NC, n_acc, D), jnp.float32),
                  jax.ShapeDtypeStruct((NC, n_acc, DGW), jnp.float32)),
        mesh=plsc.VectorSubcoreMesh(core_axis_name="c", subcore_axis_name="s"),
        scratch_types=[
            pltpu.VMEM((2, 2, CH), jnp.int32),
            pltpu.VMEM((2, 2, CH), jnp.int32),
            pltpu.VMEM((CH, D), jnp.float32),
            pltpu.VMEM((CH, D), jnp.float32),
            pltpu.VMEM((CH, DGW), jnp.float32),
            pltpu.VMEM_SHARED((n_acc, D), jnp.float32),
            pltpu.VMEM_SHARED((n_acc, DGW), jnp.float32),
            pltpu.SemaphoreType.DMA,
            pltpu.SemaphoreType.DMA,
            pltpu.SemaphoreType.DMA,
            pltpu.SemaphoreType.DMA,
        ],
        compiler_params=pltpu.CompilerParams(use_tc_tiling_on_sc=False),
    )
    def body(x_hbm, edges_hbm, out_acc, out_deg, idxG0, idxG1,
             rows0, rows1, ones_v, acc, deg, sem0, sem1, isem0, isem1):
        c = lax.axis_index("c")
        s = lax.axis_index("s")
        rpt = n_acc // NS

        def zero_rows(r, _):
            def zero_chunk(cb, _):
                rows0[r, pl.ds(cb * 16, 16)] = jnp.zeros((16,), jnp.float32)
                return 0
            lax.fori_loop(0, D // 16, zero_chunk, 0)
            ones_v[r, pl.ds(0, 16)] = jnp.zeros((16,), jnp.float32)
            return 0
        lax.fori_loop(0, CH, zero_rows, 0)

        def zero_acc(k, _):
            pltpu.sync_copy(rows0, acc.at[pl.ds(s * rpt + k * CH, CH)])
            pltpu.sync_copy(ones_v, deg.at[pl.ds(s * rpt + k * CH, CH)])
            return 0
        lax.fori_loop(0, rpt // CH, zero_acc, 0)

        one_hot = jnp.where(lax.iota(jnp.int32, 16) == 0,
                            jnp.float32(1.0), jnp.float32(0.0))

        def set_ones(r, _):
            ones_v[r, pl.ds(0, 16)] = one_hot
            return 0
        lax.fori_loop(0, CH, set_ones, 0)

        plsc.subcore_barrier()

        G = kpt // 2

        def wait_g(src_v, rows_v, sem):
            pltpu.make_async_copy(x_hbm.at[src_v], rows_v, sem).wait()

        def fire_g(src_v, rows_v, sem):
            pltpu.async_copy(x_hbm.at[src_v], rows_v, sem)

        def scat(rows_v, dst_v):
            pltpu.sync_copy(rows_v, acc.at[dst_v], add=True)
            pltpu.sync_copy(ones_v, deg.at[dst_v], add=True)

        pltpu.sync_copy(edges_hbm.at[c, s, 0], idxG0)
        fire_g(idxG0.at[0, 0], rows0, sem0)
        fire_g(idxG0.at[1, 0], rows1, sem1)
        pltpu.async_copy(edges_hbm.at[c, s, 1], idxG1, isem1)

        def quad(i, _):
            p = 2 * i + 2 < G

            wait_g(idxG0.at[0, 0], rows0, sem0)
            scat(rows0, idxG0.at[0, 1])
            pltpu.make_async_copy(edges_hbm.at[c, s, 2 * i + 1], idxG1,
                                  isem1).wait()
            fire_g(idxG1.at[0, 0], rows0, sem0)
            wait_g(idxG0.at[1, 0], rows1, sem1)
            scat(rows1, idxG0.at[1, 1])

            @pl.when(p)
            def _():
                pltpu.async_copy(edges_hbm.at[c, s, 2 * i + 2], idxG0, isem0)
            fire_g(idxG1.at[1, 0], rows1, sem1)

            wait_g(idxG1.at[0, 0], rows0, sem0)
            scat(rows0, idxG1.at[0, 1])

            @pl.when(p)
            def _():
                pltpu.make_async_copy(edges_hbm.at[c, s, 2 * i + 2], idxG0,
                                      isem0).wait()
                fire_g(idxG0.at[0, 0], rows0, sem0)
            wait_g(idxG1.at[1, 0], rows1, sem1)
            scat(rows1, idxG1.at[1, 1])

            @pl.when(p)
            def _():
                pltpu.async_copy(edges_hbm.at[c, s, 2 * i + 3], idxG1, isem1)
                fire_g(idxG0.at[1, 0], rows1, sem1)
            return 0
        lax.fori_loop(0, G // 2, quad, 0)

        plsc.subcore_barrier()
        pltpu.sync_copy(acc.at[pl.ds(s * rpt, rpt)],
                        out_acc.at[c, pl.ds(s * rpt, rpt)])
        pltpu.sync_copy(deg.at[pl.ds(s * rpt, rpt)],
                        out_deg.at[c, pl.ds(s * rpt, rpt)])

    return body(x, edges)


def _tc_body(accA_ref, accB_ref, degA_ref, degB_ref, x_ref,
             wlA_ref, wrA_ref, wlB_ref, wrB_ref,
             w1_ref, w2_ref, blA_ref, blB_ref, b1_ref, b2_ref, out_ref):
    x = x_ref[...]
    meanA = accA_ref[0] / jnp.maximum(degA_ref[0][:, :1], 1.0)
    meanB = accB_ref[0] / jnp.maximum(degB_ref[0][:, :1], 1.0)
    f32 = jnp.float32
    hA = (jnp.dot(meanA, wlA_ref[...], preferred_element_type=f32)
          + blA_ref[...]
          + jnp.dot(x, wrA_ref[...], preferred_element_type=f32))
    hB = (jnp.dot(meanB, wlB_ref[...], preferred_element_type=f32)
          + blB_ref[...]
          + jnp.dot(x, wrB_ref[...], preferred_element_type=f32))
    h = jnp.maximum(
        jnp.dot(hA, w1_ref[:D], preferred_element_type=f32)
        + jnp.dot(hB, w1_ref[D:], preferred_element_type=f32)
        + b1_ref[...], 0.0)
    out_ref[...] = jnp.dot(h, w2_ref[...], preferred_element_type=f32) + b2_ref[...]


def kernel(x_u, edge_index_uAu, edge_index_uBu, W_l_A, b_l_A, W_r_A,
           W_l_B, b_l_B, W_r_B, W1, b1, W2, b2):
    n, d = x_u.shape
    assert d == D
    e = edge_index_uAu.shape[1]
    kpt = -(-(-(-e // (NS * CH))) // 8) * 8
    ept = kpt * CH
    epad = ept * NS
    n_acc = -(-(n + 1) // (NS * CH)) * NS * CH

    pad = epad - e
    fill = jnp.concatenate([jnp.zeros((1, pad), jnp.int32),
                            jnp.full((1, pad), n, jnp.int32)], axis=0)
    edges = jnp.stack([jnp.concatenate([edge_index_uAu, fill], axis=1),
                       jnp.concatenate([edge_index_uBu, fill], axis=1)])
    edges = (edges.reshape(2, 2, NS, kpt, CH).transpose(0, 2, 3, 1, 4)
             .reshape(2, NS, kpt // 2, 2, 2, CH))

    acc, deg = _sc_segment_sums(x_u, edges, n_acc, kpt)

    grid = n // BN
    full = lambda i: (0, 0)
    out = pl.pallas_call(
        _tc_body,
        grid=(grid,),
        in_specs=[
            pl.BlockSpec((1, BN, D), lambda i: (0, i, 0)),
            pl.BlockSpec((1, BN, D), lambda i: (1, i, 0)),
            pl.BlockSpec((1, BN, DGW), lambda i: (0, i, 0)),
            pl.BlockSpec((1, BN, DGW), lambda i: (1, i, 0)),
            pl.BlockSpec((BN, D), lambda i: (i, 0)),
            pl.BlockSpec((D, D), full),
            pl.BlockSpec((D, D), full),
            pl.BlockSpec((D, D), full),
            pl.BlockSpec((D, D), full),
            pl.BlockSpec((2 * D, D), full),
            pl.BlockSpec((D, D), full),
            pl.BlockSpec((1, D), full),
            pl.BlockSpec((1, D), full),
            pl.BlockSpec((1, D), full),
            pl.BlockSpec((1, D), full),
        ],
        out_specs=pl.BlockSpec((BN, D), lambda i: (i, 0)),
        out_shape=jax.ShapeDtypeStruct((n, D), jnp.float32),
    )(acc, acc, deg, deg, x_u, W_l_A, W_r_A, W_l_B, W_r_B, W1, W2,
      b_l_A.reshape(1, D), b_l_B.reshape(1, D),
      b1.reshape(1, D), b2.reshape(1, D))
    return out


# speedup vs baseline: 2.0845x; 2.0845x over previous
import functools

import jax
import jax.numpy as jnp
from jax import lax
from jax.experimental import pallas as pl
from jax.experimental.pallas import tpu as pltpu
from jax.experimental.pallas import tpu_sc as plsc

D = 128
DGW = 16
NC = 2
NS = 16
CH = 128
BN = 1000


def _sc_segment_sums(x, edges, n_acc, kpt):
    @functools.partial(
        pl.kernel,
        out_type=(jax.ShapeDtypeStruct((NC, n_acc, D), jnp.float32),
                  jax.ShapeDtypeStruct((NC, n_acc, DGW), jnp.float32)),
        mesh=plsc.VectorSubcoreMesh(core_axis_name="c", subcore_axis_name="s"),
        scratch_types=[
            pltpu.VMEM((2, CH), jnp.int32),
            pltpu.VMEM((2, CH), jnp.int32),
            pltpu.VMEM((CH, D), jnp.float32),
            pltpu.VMEM((CH, D), jnp.float32),
            pltpu.VMEM((CH, DGW), jnp.float32),
            pltpu.VMEM_SHARED((n_acc, D), jnp.float32),
            pltpu.VMEM_SHARED((n_acc, DGW), jnp.float32),
            pltpu.SemaphoreType.DMA,
            pltpu.SemaphoreType.DMA,
        ],
        compiler_params=pltpu.CompilerParams(use_tc_tiling_on_sc=False),
    )
    def body(x_hbm, edges_hbm, out_acc, out_deg, idx0, idx1,
             rows0, rows1, ones_v, acc, deg, sem0, sem1):
        src0, dst0 = idx0.at[0], idx0.at[1]
        src1, dst1 = idx1.at[0], idx1.at[1]
        c = lax.axis_index("c")
        s = lax.axis_index("s")
        rpt = n_acc // NS

        def zero_rows(r, _):
            def zero_chunk(cb, _):
                rows0[r, pl.ds(cb * 16, 16)] = jnp.zeros((16,), jnp.float32)
                return 0
            lax.fori_loop(0, D // 16, zero_chunk, 0)
            ones_v[r, pl.ds(0, 16)] = jnp.zeros((16,), jnp.float32)
            return 0
        lax.fori_loop(0, CH, zero_rows, 0)

        def zero_acc(k, _):
            pltpu.sync_copy(rows0, acc.at[pl.ds(s * rpt + k * CH, CH)])
            pltpu.sync_copy(ones_v, deg.at[pl.ds(s * rpt + k * CH, CH)])
            return 0
        lax.fori_loop(0, rpt // CH, zero_acc, 0)

        one_hot = jnp.where(lax.iota(jnp.int32, 16) == 0,
                            jnp.float32(1.0), jnp.float32(0.0))

        def set_ones(r, _):
            ones_v[r, pl.ds(0, 16)] = one_hot
            return 0
        lax.fori_loop(0, CH, set_ones, 0)

        plsc.subcore_barrier()

        def load_idx(j, idx_v):
            pltpu.sync_copy(edges_hbm.at[c, s, j], idx_v)

        load_idx(0, idx0)
        pltpu.async_copy(x_hbm.at[src0], rows0, sem0)
        load_idx(1, idx1)
        pltpu.async_copy(x_hbm.at[src1], rows1, sem1)

        def pair(jj, _):
            j = 2 * jj
            pltpu.make_async_copy(x_hbm.at[src0], rows0, sem0).wait()
            pltpu.sync_copy(rows0, acc.at[dst0], add=True)
            pltpu.sync_copy(ones_v, deg.at[dst0], add=True)
            load_idx(j + 2, idx0)
            pltpu.async_copy(x_hbm.at[src0], rows0, sem0)
            pltpu.make_async_copy(x_hbm.at[src1], rows1, sem1).wait()
            pltpu.sync_copy(rows1, acc.at[dst1], add=True)
            pltpu.sync_copy(ones_v, deg.at[dst1], add=True)

            @pl.when(j + 3 < kpt)
            def _():
                load_idx(j + 3, idx1)
                pltpu.async_copy(x_hbm.at[src1], rows1, sem1)
            return 0
        lax.fori_loop(0, (kpt - 1) // 2, pair, 0)

        pltpu.make_async_copy(x_hbm.at[src0], rows0, sem0).wait()
        pltpu.sync_copy(rows0, acc.at[dst0], add=True)
        pltpu.sync_copy(ones_v, deg.at[dst0], add=True)

        plsc.subcore_barrier()
        pltpu.sync_copy(acc.at[pl.ds(s * rpt, rpt)],
                        out_acc.at[c, pl.ds(s * rpt, rpt)])
        pltpu.sync_copy(deg.at[pl.ds(s * rpt, rpt)],
                        out_deg.at[c, pl.ds(s * rpt, rpt)])

    return body(x, edges)


def _tc_body(accA_ref, accB_ref, degA_ref, degB_ref, x_ref,
             wlA_ref, wrA_ref, wlB_ref, wrB_ref,
             w1_ref, w2_ref, blA_ref, blB_ref, b1_ref, b2_ref, out_ref):
    x = x_ref[...]
    meanA = accA_ref[0] / jnp.maximum(degA_ref[0][:, :1], 1.0)
    meanB = accB_ref[0] / jnp.maximum(degB_ref[0][:, :1], 1.0)
    f32 = jnp.float32
    hA = (jnp.dot(meanA, wlA_ref[...], preferred_element_type=f32)
          + blA_ref[...]
          + jnp.dot(x, wrA_ref[...], preferred_element_type=f32))
    hB = (jnp.dot(meanB, wlB_ref[...], preferred_element_type=f32)
          + blB_ref[...]
          + jnp.dot(x, wrB_ref[...], preferred_element_type=f32))
    h = jnp.maximum(
        jnp.dot(hA, w1_ref[:D], preferred_element_type=f32)
        + jnp.dot(hB, w1_ref[D:], preferred_element_type=f32)
        + b1_ref[...], 0.0)
    out_ref[...] = jnp.dot(h, w2_ref[...], preferred_element_type=f32) + b2_ref[...]


def kernel(x_u, edge_index_uAu, edge_index_uBu, W_l_A, b_l_A, W_r_A,
           W_l_B, b_l_B, W_r_B, W1, b1, W2, b2):
    n, d = x_u.shape
    assert d == D
    e = edge_index_uAu.shape[1]
    kpt = -(-e // (NS * CH))
    assert kpt % 2 == 1
    ept = kpt * CH
    epad = ept * NS
    n_acc = -(-(n + 1) // (NS * CH)) * NS * CH

    pad = epad - e
    fill = jnp.concatenate([jnp.zeros((1, pad), jnp.int32),
                            jnp.full((1, pad), n, jnp.int32)], axis=0)
    edges = jnp.stack([jnp.concatenate([edge_index_uAu, fill], axis=1),
                       jnp.concatenate([edge_index_uBu, fill], axis=1)])
    edges = edges.reshape(2, 2, NS, kpt, CH).transpose(0, 2, 3, 1, 4)

    acc, deg = _sc_segment_sums(x_u, edges, n_acc, kpt)

    grid = n // BN
    full = lambda i: (0, 0)
    out = pl.pallas_call(
        _tc_body,
        grid=(grid,),
        in_specs=[
            pl.BlockSpec((1, BN, D), lambda i: (0, i, 0)),
            pl.BlockSpec((1, BN, D), lambda i: (1, i, 0)),
            pl.BlockSpec((1, BN, DGW), lambda i: (0, i, 0)),
            pl.BlockSpec((1, BN, DGW), lambda i: (1, i, 0)),
            pl.BlockSpec((BN, D), lambda i: (i, 0)),
            pl.BlockSpec((D, D), full),
            pl.BlockSpec((D, D), full),
            pl.BlockSpec((D, D), full),
            pl.BlockSpec((D, D), full),
            pl.BlockSpec((2 * D, D), full),
            pl.BlockSpec((D, D), full),
            pl.BlockSpec((1, D), full),
            pl.BlockSpec((1, D), full),
            pl.BlockSpec((1, D), full),
            pl.BlockSpec((1, D), full),
        ],
        out_specs=pl.BlockSpec((BN, D), lambda i: (i, 0)),
        out_shape=jax.ShapeDtypeStruct((n, D), jnp.float32),
    )(acc, acc, deg, deg, x_u, W_l_A, W_r_A, W_l_B, W_r_B, W1, W2,
      b_l_A.reshape(1, D), b_l_B.reshape(1, D),
      b1.reshape(1, D), b2.reshape(1, D))
    return out
